# baseline (speedup 1.0000x reference)
import functools

import jax
import jax.numpy as jnp
from jax.experimental import pallas as pl
from jax.experimental.pallas import tpu as pltpu

B = 16384
D = 64
R = 128
NSTEPS = B // R


def _tc_body(users_s, items_s, ut, it, out, buf_u, buf_i, sem_u, sem_i):
  i = pl.program_id(0)

  def issue(step, buf):
    base = step * R
    for l in range(R):
      pltpu.make_async_copy(ut.at[users_s[base + l]], buf_u.at[buf, l],
                            sem_u).start()
      pltpu.make_async_copy(it.at[items_s[base + l]], buf_i.at[buf, l],
                            sem_i).start()

  @pl.when(i == 0)
  def _():
    issue(0, 0)

  @pl.when(i + 1 < NSTEPS)
  def _():
    issue(i + 1, (i + 1) % 2)

  cur = i % 2
  pltpu.make_async_copy(ut.at[pl.ds(0, R)], buf_u.at[cur], sem_u).wait()
  pltpu.make_async_copy(it.at[pl.ds(0, R)], buf_i.at[cur], sem_i).wait()

  u = buf_u[cur]
  v = buf_i[cur]
  out[...] = jnp.sum(u * v, axis=1)


@jax.jit
def kernel(users, items, user_table, item_table):
  grid_spec = pltpu.PrefetchScalarGridSpec(
      num_scalar_prefetch=2,
      grid=(NSTEPS,),
      in_specs=[
          pl.BlockSpec(memory_space=pl.ANY),
          pl.BlockSpec(memory_space=pl.ANY),
      ],
      out_specs=pl.BlockSpec((R,), lambda i, *_: (i,)),
      scratch_shapes=[
          pltpu.VMEM((2, R, D), jnp.float32),
          pltpu.VMEM((2, R, D), jnp.float32),
          pltpu.SemaphoreType.DMA,
          pltpu.SemaphoreType.DMA,
      ],
  )
  return pl.pallas_call(
      _tc_body,
      grid_spec=grid_spec,
      out_shape=jax.ShapeDtypeStruct((B,), jnp.float32),
      compiler_params=pltpu.CompilerParams(
          dimension_semantics=("arbitrary",)),
  )(users, items, user_table, item_table)


# speedup vs baseline: 1.0013x; 1.0013x over previous
import jax
import jax.numpy as jnp
from jax.experimental import pallas as pl
from jax.experimental.pallas import tpu as pltpu

B = 16384
D = 64
R = 128
NSTEPS = B // R
K = 4
RG = R // K


def _tc_body(users_s, items_s, ut, it, out, buf_u, buf_i, *sems):
  i = pl.program_id(0)

  def issue(step, buf, parity):
    base = step * R
    for g in range(K):
      su = sems[(0 * 2 + parity) * K + g]
      si = sems[(1 * 2 + parity) * K + g]
      for l in range(g * RG, (g + 1) * RG):
        pltpu.make_async_copy(ut.at[users_s[base + l]], buf_u.at[buf, l],
                              su).start()
        pltpu.make_async_copy(it.at[items_s[base + l]], buf_i.at[buf, l],
                              si).start()

  def drain(buf, parity):
    for g in range(K):
      pltpu.make_async_copy(ut.at[pl.ds(0, RG)],
                            buf_u.at[buf, pl.ds(g * RG, RG)],
                            sems[(0 * 2 + parity) * K + g]).wait()
      pltpu.make_async_copy(it.at[pl.ds(0, RG)],
                            buf_i.at[buf, pl.ds(g * RG, RG)],
                            sems[(1 * 2 + parity) * K + g]).wait()

  @pl.when(i == 0)
  def _():
    issue(0, 0, 0)

  nxt = i + 1

  @pl.when(jnp.logical_and(nxt < NSTEPS, nxt % 2 == 0))
  def _():
    issue(nxt, nxt % 2, 0)

  @pl.when(jnp.logical_and(nxt < NSTEPS, nxt % 2 == 1))
  def _():
    issue(nxt, nxt % 2, 1)

  cur = i % 2

  @pl.when(i % 2 == 0)
  def _():
    drain(cur, 0)

  @pl.when(i % 2 == 1)
  def _():
    drain(cur, 1)

  u = buf_u[cur]
  v = buf_i[cur]
  out[...] = jnp.sum(u * v, axis=1)


@jax.jit
def kernel(users, items, user_table, item_table):
  grid_spec = pltpu.PrefetchScalarGridSpec(
      num_scalar_prefetch=2,
      grid=(NSTEPS,),
      in_specs=[
          pl.BlockSpec(memory_space=pl.ANY),
          pl.BlockSpec(memory_space=pl.ANY),
      ],
      out_specs=pl.BlockSpec((R,), lambda i, *_: (i,)),
      scratch_shapes=[
          pltpu.VMEM((2, R, D), jnp.float32),
          pltpu.VMEM((2, R, D), jnp.float32),
      ] + [pltpu.SemaphoreType.DMA] * (2 * 2 * K),
  )
  return pl.pallas_call(
      _tc_body,
      grid_spec=grid_spec,
      out_shape=jax.ShapeDtypeStruct((B,), jnp.float32),
      compiler_params=pltpu.CompilerParams(
          dimension_semantics=("arbitrary",)),
  )(users, items, user_table, item_table)


# speedup vs baseline: 1.0639x; 1.0625x over previous
import jax
import jax.numpy as jnp
from jax import lax
from jax.experimental import pallas as pl
from jax.experimental.pallas import tpu as pltpu
from jax.experimental.pallas import tpu_sc as plsc

B = 16384
D = 64
L = 16
NC = 2
NS = 16
NW = NC * NS

S_SC = 8192
S_TC = B - S_SC


SC_PER_W = S_SC // NW


def _sc_body(users, items, user_table, item_table, out,
             idx_u_v, idx_i_v, rows_u, rows_i, out_v, sem_g):
  wid = lax.axis_index("s") * NC + lax.axis_index("c")
  base = wid * SC_PER_W

  pltpu.sync_copy(users.at[pl.ds(base, SC_PER_W)], idx_u_v)
  pltpu.sync_copy(items.at[pl.ds(base, SC_PER_W)], idx_i_v)

  riota = lax.iota(jnp.int32, L)

  def issue(g, carry):
    uvec = idx_u_v[pl.ds(g * L, L)]
    ivec = idx_i_v[pl.ds(g * L, L)]
    for l in range(L):
      pltpu.async_copy(user_table.at[uvec[l]], rows_u.at[g * L + l], sem_g)
      pltpu.async_copy(item_table.at[ivec[l]], rows_i.at[g * L + l], sem_g)
    return carry

  lax.fori_loop(0, SC_PER_W // L, issue, 0)

  def drain(r, carry):
    pltpu.make_async_copy(user_table.at[0], rows_u.at[0], sem_g).wait()
    pltpu.make_async_copy(item_table.at[0], rows_i.at[0], sem_g).wait()
    return carry

  lax.fori_loop(0, SC_PER_W, drain, 0)

  def block(j, carry):
    ro = j * L
    row_ids = riota + ro
    acc = jnp.zeros((L,), jnp.float32)
    for k in range(D):
      col = jnp.full((L,), k, jnp.int32)
      uv = plsc.load_gather(rows_u, [row_ids, col])
      iv = plsc.load_gather(rows_i, [row_ids, col])
      acc = acc + uv * iv
    out_v[pl.ds(ro, L)] = acc
    return carry

  lax.fori_loop(0, SC_PER_W // L, block, 0)

  pltpu.sync_copy(out_v, out.at[pl.ds(base, SC_PER_W)])


def _sc_call(users, items, user_table, item_table):
  mesh = plsc.VectorSubcoreMesh(core_axis_name="c", subcore_axis_name="s")
  k = pl.kernel(
      _sc_body,
      out_type=jax.ShapeDtypeStruct((S_SC,), jnp.float32),
      mesh=mesh,
      scratch_types=[
          pltpu.VMEM((SC_PER_W,), jnp.int32),
          pltpu.VMEM((SC_PER_W,), jnp.int32),
          pltpu.VMEM((SC_PER_W, D), jnp.float32),
          pltpu.VMEM((SC_PER_W, D), jnp.float32),
          pltpu.VMEM((SC_PER_W,), jnp.float32),
          pltpu.SemaphoreType.DMA,
      ],
      compiler_params=pltpu.CompilerParams(needs_layout_passes=False),
  )
  return k(users, items, user_table, item_table)


R = 128
NSTEPS = S_TC // R
K = 4
RG = R // K


def _tc_body(users_s, items_s, ut, it, out, buf_u, buf_i, *sems):
  i = pl.program_id(0)

  def issue(step, buf, parity):
    base = step * R
    for g in range(K):
      su = sems[(0 * 2 + parity) * K + g]
      si = sems[(1 * 2 + parity) * K + g]
      for l in range(g * RG, (g + 1) * RG):
        pltpu.make_async_copy(ut.at[users_s[base + l]], buf_u.at[buf, l],
                              su).start()
        pltpu.make_async_copy(it.at[items_s[base + l]], buf_i.at[buf, l],
                              si).start()

  def drain(buf, parity):
    for g in range(K):
      pltpu.make_async_copy(ut.at[pl.ds(0, RG)],
                            buf_u.at[buf, pl.ds(g * RG, RG)],
                            sems[(0 * 2 + parity) * K + g]).wait()
      pltpu.make_async_copy(it.at[pl.ds(0, RG)],
                            buf_i.at[buf, pl.ds(g * RG, RG)],
                            sems[(1 * 2 + parity) * K + g]).wait()

  @pl.when(i == 0)
  def _():
    issue(0, 0, 0)

  nxt = i + 1

  @pl.when(jnp.logical_and(nxt < NSTEPS, nxt % 2 == 0))
  def _():
    issue(nxt, nxt % 2, 0)

  @pl.when(jnp.logical_and(nxt < NSTEPS, nxt % 2 == 1))
  def _():
    issue(nxt, nxt % 2, 1)

  cur = i % 2

  @pl.when(i % 2 == 0)
  def _():
    drain(cur, 0)

  @pl.when(i % 2 == 1)
  def _():
    drain(cur, 1)

  u = buf_u[cur]
  v = buf_i[cur]
  out[...] = jnp.sum(u * v, axis=1)


def _tc_call(users, items, user_table, item_table):
  grid_spec = pltpu.PrefetchScalarGridSpec(
      num_scalar_prefetch=2,
      grid=(NSTEPS,),
      in_specs=[
          pl.BlockSpec(memory_space=pl.ANY),
          pl.BlockSpec(memory_space=pl.ANY),
      ],
      out_specs=pl.BlockSpec((R,), lambda i, *_: (i,)),
      scratch_shapes=[
          pltpu.VMEM((2, R, D), jnp.float32),
          pltpu.VMEM((2, R, D), jnp.float32),
      ] + [pltpu.SemaphoreType.DMA] * (2 * 2 * K),
  )
  return pl.pallas_call(
      _tc_body,
      grid_spec=grid_spec,
      out_shape=jax.ShapeDtypeStruct((S_TC,), jnp.float32),
      compiler_params=pltpu.CompilerParams(
          dimension_semantics=("arbitrary",)),
  )(users, items, user_table, item_table)


@jax.jit
def kernel(users, items, user_table, item_table):
  out_sc = _sc_call(users[:S_SC], items[:S_SC], user_table, item_table)
  out_tc = _tc_call(users[S_SC:], items[S_SC:], user_table, item_table)
  return jnp.concatenate([out_sc, out_tc])


# speedup vs baseline: 1.0657x; 1.0017x over previous
import jax
import jax.numpy as jnp
from jax import lax
from jax.experimental import pallas as pl
from jax.experimental.pallas import tpu as pltpu
from jax.experimental.pallas import tpu_sc as plsc

B = 16384
D = 64
L = 16
NC = 2
NS = 16
NW = NC * NS

S_SC = 8192
S_TC = B - S_SC


SC_PER_W = S_SC // NW


def _sc_body(users, items, user_table, item_table, out,
             idx_u_v, idx_i_v, rows_u, rows_i, out_v, sem_g):
  wid = lax.axis_index("s") * NC + lax.axis_index("c")
  base = wid * SC_PER_W

  pltpu.sync_copy(users.at[pl.ds(base, SC_PER_W)], idx_u_v)
  pltpu.sync_copy(items.at[pl.ds(base, SC_PER_W)], idx_i_v)

  riota = lax.iota(jnp.int32, L)

  def issue(g, carry):
    uvec = idx_u_v[pl.ds(g * L, L)]
    ivec = idx_i_v[pl.ds(g * L, L)]
    for l in range(L):
      pltpu.async_copy(user_table.at[uvec[l]], rows_u.at[g * L + l], sem_g)
      pltpu.async_copy(item_table.at[ivec[l]], rows_i.at[g * L + l], sem_g)
    return carry

  lax.fori_loop(0, SC_PER_W // L, issue, 0)

  def drain(r, carry):
    pltpu.make_async_copy(user_table.at[0], rows_u.at[0], sem_g).wait()
    pltpu.make_async_copy(item_table.at[0], rows_i.at[0], sem_g).wait()
    return carry

  lax.fori_loop(0, SC_PER_W, drain, 0)

  def block(j, carry):
    ro = j * L
    row_ids = riota + ro
    acc = jnp.zeros((L,), jnp.float32)
    for k in range(D):
      col = jnp.full((L,), k, jnp.int32)
      uv = plsc.load_gather(rows_u, [row_ids, col])
      iv = plsc.load_gather(rows_i, [row_ids, col])
      acc = acc + uv * iv
    out_v[pl.ds(ro, L)] = acc
    return carry

  lax.fori_loop(0, SC_PER_W // L, block, 0)

  pltpu.sync_copy(out_v, out.at[pl.ds(base, SC_PER_W)])


def _sc_call(users, items, user_table, item_table):
  mesh = plsc.VectorSubcoreMesh(core_axis_name="c", subcore_axis_name="s")
  k = pl.kernel(
      _sc_body,
      out_type=jax.ShapeDtypeStruct((S_SC,), jnp.float32),
      mesh=mesh,
      scratch_types=[
          pltpu.VMEM((SC_PER_W,), jnp.int32),
          pltpu.VMEM((SC_PER_W,), jnp.int32),
          pltpu.VMEM((SC_PER_W, D), jnp.float32),
          pltpu.VMEM((SC_PER_W, D), jnp.float32),
          pltpu.VMEM((SC_PER_W,), jnp.float32),
          pltpu.SemaphoreType.DMA,
      ],
      compiler_params=pltpu.CompilerParams(needs_layout_passes=False),
  )
  return k(users, items, user_table, item_table)


R = 128
NSTEPS = S_TC // R
K = 4
RG = R // K


def _tc_body(users_s, items_s, ut, it, out, buf_u, buf_i, *sems):
  i = pl.program_id(0)

  def issue(step, buf, parity):
    base = step * R
    for g in range(K):
      su = sems[(0 * 2 + parity) * K + g]
      si = sems[(1 * 2 + parity) * K + g]
      for l in range(g * RG, (g + 1) * RG):
        pltpu.make_async_copy(ut.at[users_s[base + l]], buf_u.at[buf, l],
                              su).start()
        pltpu.make_async_copy(it.at[items_s[base + l]], buf_i.at[buf, l],
                              si).start()

  def drain(buf, parity):
    for g in range(K):
      pltpu.make_async_copy(ut.at[pl.ds(0, RG)],
                            buf_u.at[buf, pl.ds(g * RG, RG)],
                            sems[(0 * 2 + parity) * K + g]).wait()
      pltpu.make_async_copy(it.at[pl.ds(0, RG)],
                            buf_i.at[buf, pl.ds(g * RG, RG)],
                            sems[(1 * 2 + parity) * K + g]).wait()

  @pl.when(i == 0)
  def _():
    issue(0, 0, 0)

  nxt = i + 1

  @pl.when(jnp.logical_and(nxt < NSTEPS, nxt % 2 == 0))
  def _():
    issue(nxt, nxt % 2, 0)

  @pl.when(jnp.logical_and(nxt < NSTEPS, nxt % 2 == 1))
  def _():
    issue(nxt, nxt % 2, 1)

  cur = i % 2

  @pl.when(i % 2 == 0)
  def _():
    drain(cur, 0)

  @pl.when(i % 2 == 1)
  def _():
    drain(cur, 1)

  u = buf_u[cur]
  v = buf_i[cur]
  out[...] = jnp.sum(u * v, axis=1)


def _tc_call(users, items, user_table, item_table):
  grid_spec = pltpu.PrefetchScalarGridSpec(
      num_scalar_prefetch=2,
      grid=(NSTEPS,),
      in_specs=[
          pl.BlockSpec(memory_space=pl.ANY),
          pl.BlockSpec(memory_space=pl.ANY),
      ],
      out_specs=pl.BlockSpec((R,), lambda i, *_: (i,)),
      scratch_shapes=[
          pltpu.VMEM((2, R, D), jnp.float32),
          pltpu.VMEM((2, R, D), jnp.float32),
      ] + [pltpu.SemaphoreType.DMA] * (2 * 2 * K),
  )
  return pl.pallas_call(
      _tc_body,
      grid_spec=grid_spec,
      out_shape=jax.ShapeDtypeStruct((S_TC,), jnp.float32),
      compiler_params=pltpu.CompilerParams(
          dimension_semantics=("arbitrary",)),
  )(users, items, user_table, item_table)


@jax.jit
def kernel(users, items, user_table, item_table):
  out_tc = _tc_call(users[S_SC:], items[S_SC:], user_table, item_table)
  out_sc = _sc_call(users[:S_SC], items[:S_SC], user_table, item_table)
  return jnp.concatenate([out_sc, out_tc])


# speedup vs baseline: 1.1050x; 1.0368x over previous
import jax
import jax.numpy as jnp
from jax import lax
from jax.experimental import pallas as pl
from jax.experimental.pallas import tpu as pltpu
from jax.experimental.pallas import tpu_sc as plsc

B = 16384
D = 64
L = 16
NC = 2
NS = 16
NW = NC * NS
B_PER_W = B // NW
PASS_ROWS = B_PER_W // 2


def _body(users, items, user_table, item_table, out,
          idx_u_v, idx_i_v, rows_u, rows_i, out_v, sem_g):
  wid = lax.axis_index("s") * NC + lax.axis_index("c")
  base = wid * B_PER_W

  pltpu.sync_copy(users.at[pl.ds(base, B_PER_W)], idx_u_v)
  pltpu.sync_copy(items.at[pl.ds(base, B_PER_W)], idx_i_v)

  riota = lax.iota(jnp.int32, L)

  for p in range(2):
    poff = p * PASS_ROWS

    def issue(g, carry, poff=poff):
      uvec = idx_u_v[pl.ds(poff + g * L, L)]
      ivec = idx_i_v[pl.ds(poff + g * L, L)]
      for l in range(L):
        pltpu.async_copy(user_table.at[uvec[l]], rows_u.at[g * L + l], sem_g)
        pltpu.async_copy(item_table.at[ivec[l]], rows_i.at[g * L + l], sem_g)
      return carry

    lax.fori_loop(0, PASS_ROWS // L, issue, 0)

    def drain(r, carry):
      pltpu.make_async_copy(user_table.at[0], rows_u.at[0], sem_g).wait()
      pltpu.make_async_copy(item_table.at[0], rows_i.at[0], sem_g).wait()
      return carry

    lax.fori_loop(0, PASS_ROWS, drain, 0)

    def block(j, carry):
      ro = j * L
      row_ids = riota + ro
      acc = jnp.zeros((L,), jnp.float32)
      for k in range(D):
        col = jnp.full((L,), k, jnp.int32)
        uv = plsc.load_gather(rows_u, [row_ids, col])
        iv = plsc.load_gather(rows_i, [row_ids, col])
        acc = acc + uv * iv
      out_v[pl.ds(ro, L)] = acc
      return carry

    lax.fori_loop(0, PASS_ROWS // L, block, 0)

    pltpu.sync_copy(out_v, out.at[pl.ds(base + poff, PASS_ROWS)])


@jax.jit
def kernel(users, items, user_table, item_table):
  mesh = plsc.VectorSubcoreMesh(core_axis_name="c", subcore_axis_name="s")
  k = pl.kernel(
      _body,
      out_type=jax.ShapeDtypeStruct((B,), jnp.float32),
      mesh=mesh,
      scratch_types=[
          pltpu.VMEM((B_PER_W,), jnp.int32),
          pltpu.VMEM((B_PER_W,), jnp.int32),
          pltpu.VMEM((PASS_ROWS, D), jnp.float32),
          pltpu.VMEM((PASS_ROWS, D), jnp.float32),
          pltpu.VMEM((PASS_ROWS,), jnp.float32),
          pltpu.SemaphoreType.DMA,
      ],
      compiler_params=pltpu.CompilerParams(needs_layout_passes=False),
  )
  return k(users, items, user_table, item_table)
